# baseline (speedup 1.0000x reference)
import functools

import jax
import jax.numpy as jnp
from jax import lax
from jax.experimental import pallas as pl
from jax.experimental.pallas import tpu as pltpu
from jax.experimental.pallas import tpu_sc as plsc

N = 320000
D = 128
S = 10000
NC = 2
SH = S // NC
NS = 16
CHUNK = 128
NCHUNKS = N // CHUNK
NBUF = 4
AROWS = SH + 8

_mesh = plsc.VectorSubcoreMesh(core_axis_name="c", subcore_axis_name="s")


@functools.partial(
    pl.kernel,
    out_type=jax.ShapeDtypeStruct((S, D), jnp.float32),
    mesh=_mesh,
    scratch_types=[
        pltpu.VMEM((NBUF, CHUNK, D), jnp.float32),
        pltpu.VMEM((NBUF, 128), jnp.int32),
        pltpu.VMEM((16,), jnp.int32),
        pltpu.VMEM((16, D), jnp.float32),
        pltpu.VMEM((16,), jnp.int32),
        pltpu.VMEM_SHARED((AROWS, D), jnp.float32),
        pltpu.SemaphoreType.DMA,
        pltpu.SemaphoreType.DMA,
        pltpu.SemaphoreType.DMA,
        pltpu.SemaphoreType.DMA,
        pltpu.SemaphoreType.DMA,
    ],
)
def _seg_sum_sc(feat_hbm, idx_hbm, out_hbm,
                feat_bufs, idx_bufs, pbuf, local_acc, cidx, acc,
                gsem0, gsem1, gsem2, gsem3, zsem):
    c = lax.axis_index("c")
    s = lax.axis_index("s")
    gsems = (gsem0, gsem1, gsem2, gsem3)

    zero16 = jnp.zeros((16,), jnp.float32)

    def zrow(i, carry):
        for j in range(D // 16):
            feat_bufs[0, i, pl.ds(j * 16, 16)] = zero16
        return carry

    lax.fori_loop(0, CHUNK, zrow, 0)
    zbuf = feat_bufs.at[0]
    off = pl.multiple_of(s * (AROWS // NS) - (s % 8), 8)
    zcopies = [(zbuf, acc.at[pl.ds(pl.multiple_of(off + z * CHUNK, 8), CHUNK)])
               for z in range(312 // CHUNK)]
    zrem = 312 % CHUNK
    if zrem:
        zcopies.append(
            (zbuf.at[pl.ds(0, zrem)],
             acc.at[pl.ds(pl.multiple_of(off + 312 - zrem, 8), zrem)]))
    for src, dst in zcopies:
        pltpu.async_copy(src, dst, zsem)

    @pl.when(s % 8 == 7)
    def _zero_tail():
        pltpu.async_copy(zbuf.at[pl.ds(0, 8)],
                         acc.at[pl.ds(pl.multiple_of(off + 312, 8), 8)], zsem)

    def probe(_, lohi):
        lo, hi = lohi
        mid = (lo + hi) // 2

        def probed():
            pltpu.sync_copy(idx_hbm.at[pl.ds(pl.multiple_of(mid * CHUNK, 8),
                                             16)], pbuf)
            v = pbuf[...][0]
            ge = (v >= SH).astype(jnp.int32)
            return ge * mid + (1 - ge) * hi, ge * lo + (1 - ge) * (mid + 1)

        def done():
            return hi, lo

        hi2, lo2 = lax.cond(lo < hi, probed, done)
        return lo2, hi2

    lo0, hi0 = jnp.int32(0), jnp.int32(NCHUNKS)
    r_lo, r_hi = lax.fori_loop(0, 12, probe, (lo0, hi0))
    r = r_hi
    nc0 = r
    nc1 = jnp.maximum(r - 1, 0)

    for src, dst in zcopies:
        pltpu.make_async_copy(src, dst, zsem).wait()

    @pl.when(s % 8 == 7)
    def _zero_tail_wait():
        pltpu.make_async_copy(
            zbuf.at[pl.ds(0, 8)],
            acc.at[pl.ds(pl.multiple_of(off + 312, 8), 8)], zsem).wait()

    mybase = c * nc1
    mycnt = (1 - c) * nc0 + c * (NCHUNKS - nc1)
    q = (mycnt + NS - 1) // NS
    lo = s * q
    qs = jnp.minimum(jnp.maximum(mycnt - lo, 0), q)

    plsc.subcore_barrier()

    rowbase = c * SH

    def issue_gather(b, k):
        rbase = pl.multiple_of((mybase + lo + k) * CHUNK, 8)
        pltpu.async_copy(feat_hbm.at[pl.ds(rbase, CHUNK)],
                         feat_bufs.at[b], gsems[b])
        pltpu.async_copy(idx_hbm.at[pl.ds(rbase, CHUNK)],
                         idx_bufs.at[b], gsems[b])

    def wait_gather(b, k):
        rbase = pl.multiple_of((mybase + lo + k) * CHUNK, 8)
        pltpu.make_async_copy(feat_hbm.at[pl.ds(rbase, CHUNK)],
                              feat_bufs.at[b], gsems[b]).wait()
        pltpu.make_async_copy(idx_hbm.at[pl.ds(rbase, CHUNK)],
                              idx_bufs.at[b], gsems[b]).wait()

    def lzrow(i, carry):
        for j in range(D // 16):
            local_acc[i, pl.ds(j * 16, 16)] = zero16
        return carry

    lax.fori_loop(0, 16, lzrow, 0)
    lane = lax.iota(jnp.int32, 16)
    trash16 = jnp.full((16,), SH, jnp.int32)

    for b in range(NBUF):
        @pl.when(b < qs)
        def _prime():
            issue_gather(b, b)

    def superstep(i, carry):
        for b in range(NBUF):
            k = i * NBUF + b

            @pl.when(k < qs)
            def _do():
                wait_gather(b, k)

                def grp(g, carry2):
                    prev, rank, cvec = carry2
                    iv = idx_bufs[b, pl.ds(pl.multiple_of(g * 16, 16), 16)]
                    for j in range(16):
                        sidx = iv[j]
                        rank = rank + (sidx != prev).astype(jnp.int32)
                        t = sidx - rowbase
                        oob = (t < 0) | (t >= SH)
                        tg = jnp.where(oob, SH, t)
                        cvec = jnp.where(lane == rank,
                                         jnp.full((16,), tg, jnp.int32), cvec)
                        rk = jnp.minimum(rank, 15)
                        i2 = g * 16 + j
                        for j2 in range(D // 16):
                            plsc.addupdate(
                                local_acc.at[rk, pl.ds(j2 * 16, 16)],
                                feat_bufs[b, i2, pl.ds(j2 * 16, 16)])
                        prev = sidx
                    return prev, rank, cvec

                _, nseg1, cvec = lax.fori_loop(
                    0, CHUNK // 16, grp,
                    (jnp.int32(-1), jnp.int32(-1), trash16))
                nseg = nseg1 + 1

                @pl.when(nseg <= 16)
                def _flush_compact():
                    cidx[...] = cvec
                    pltpu.sync_copy(local_acc, acc.at[cidx], add=True)

                @pl.when(nseg > 16)
                def _flush_raw():
                    for j in range(CHUNK // 16):
                        v = idx_bufs[b, pl.ds(j * 16, 16)] - rowbase
                        oob = (v < 0) | (v >= SH)
                        idx_bufs[b, pl.ds(j * 16, 16)] = jnp.where(
                            oob, trash16, v)
                    pltpu.sync_copy(feat_bufs.at[b], acc.at[idx_bufs.at[b]],
                                    add=True)

                @pl.when(k + NBUF < qs)
                def _prefetch():
                    issue_gather(b, k + NBUF)

                def lz(r2, carry2):
                    for j in range(D // 16):
                        local_acc[r2, pl.ds(j * 16, 16)] = zero16
                    return carry2

                lax.fori_loop(0, jnp.minimum(nseg, 16), lz, 0)
        return carry

    lax.fori_loop(0, (q + NBUF - 1) // NBUF, superstep, 0)
    plsc.subcore_barrier()

    obase = pl.multiple_of(c * SH + off, 8)
    pltpu.sync_copy(acc.at[pl.ds(off, 312)], out_hbm.at[pl.ds(obase, 312)])

    @pl.when((s % 8 == 7) & (s < NS - 1))
    def _dump_tail():
        pltpu.sync_copy(acc.at[pl.ds(pl.multiple_of(off + 312, 8), 8)],
                        out_hbm.at[pl.ds(pl.multiple_of(c * SH + off + 312, 8), 8)])


@jax.jit
def kernel(features, structural_indices):
    return _seg_sum_sc(features, structural_indices)


# speedup vs baseline: 2.9161x; 2.9161x over previous
import functools

import jax
import jax.numpy as jnp
from jax import lax
from jax.experimental import pallas as pl
from jax.experimental.pallas import tpu as pltpu
from jax.experimental.pallas import tpu_sc as plsc

N = 320000
D = 128
S = 10000
NC = 2
SH = S // NC
NS = 16
CHUNK = 128
NCHUNKS = N // CHUNK
NBUF = 4
AROWS = SH + 8

_mesh = plsc.VectorSubcoreMesh(core_axis_name="c", subcore_axis_name="s")


@functools.partial(
    pl.kernel,
    out_type=jax.ShapeDtypeStruct((S, D), jnp.float32),
    mesh=_mesh,
    scratch_types=[
        pltpu.VMEM((NBUF, CHUNK, D), jnp.float32),
        pltpu.VMEM((NBUF, 128), jnp.int32),
        pltpu.VMEM((16,), jnp.int32),
        pltpu.VMEM_SHARED((AROWS, D), jnp.float32),
        pltpu.SemaphoreType.DMA,
        pltpu.SemaphoreType.DMA,
        pltpu.SemaphoreType.DMA,
        pltpu.SemaphoreType.DMA,
        pltpu.SemaphoreType.DMA,
    ],
)
def _seg_sum_sc(feat_hbm, idx_hbm, out_hbm,
                feat_bufs, idx_bufs, pbuf, acc,
                gsem0, gsem1, gsem2, gsem3, zsem):
    c = lax.axis_index("c")
    s = lax.axis_index("s")
    gsems = (gsem0, gsem1, gsem2, gsem3)

    zero16 = jnp.zeros((16,), jnp.float32)

    def zrow(i, carry):
        for j in range(D // 16):
            feat_bufs[0, i, pl.ds(j * 16, 16)] = zero16
        return carry

    lax.fori_loop(0, CHUNK, zrow, 0)
    zbuf = feat_bufs.at[0]
    off = pl.multiple_of(s * (AROWS // NS) - (s % 8), 8)
    zcopies = [(zbuf, acc.at[pl.ds(pl.multiple_of(off + z * CHUNK, 8), CHUNK)])
               for z in range(312 // CHUNK)]
    zrem = 312 % CHUNK
    if zrem:
        zcopies.append(
            (zbuf.at[pl.ds(0, zrem)],
             acc.at[pl.ds(pl.multiple_of(off + 312 - zrem, 8), zrem)]))
    for src, dst in zcopies:
        pltpu.async_copy(src, dst, zsem)

    @pl.when(s % 8 == 7)
    def _zero_tail():
        pltpu.async_copy(zbuf.at[pl.ds(0, 8)],
                         acc.at[pl.ds(pl.multiple_of(off + 312, 8), 8)], zsem)

    def probe(_, lohi):
        lo, hi = lohi
        mid = (lo + hi) // 2

        def probed():
            pltpu.sync_copy(idx_hbm.at[pl.ds(pl.multiple_of(mid * CHUNK, 8),
                                             16)], pbuf)
            v = pbuf[...][0]
            ge = (v >= SH).astype(jnp.int32)
            return ge * mid + (1 - ge) * hi, ge * lo + (1 - ge) * (mid + 1)

        def done():
            return hi, lo

        hi2, lo2 = lax.cond(lo < hi, probed, done)
        return lo2, hi2

    lo0, hi0 = jnp.int32(0), jnp.int32(NCHUNKS)
    r_lo, r_hi = lax.fori_loop(0, 12, probe, (lo0, hi0))
    r = r_hi
    nc0 = r
    nc1 = jnp.maximum(r - 1, 0)

    for src, dst in zcopies:
        pltpu.make_async_copy(src, dst, zsem).wait()

    @pl.when(s % 8 == 7)
    def _zero_tail_wait():
        pltpu.make_async_copy(
            zbuf.at[pl.ds(0, 8)],
            acc.at[pl.ds(pl.multiple_of(off + 312, 8), 8)], zsem).wait()
    mybase = c * nc1
    mycnt = (1 - c) * nc0 + c * (NCHUNKS - nc1)
    q = (mycnt + NS - 1) // NS
    lo = s * q
    qs = jnp.minimum(jnp.maximum(mycnt - lo, 0), q)

    plsc.subcore_barrier()

    rowbase = c * SH

    def issue_gather(b, k):
        rbase = pl.multiple_of((mybase + lo + k) * CHUNK, 8)
        pltpu.async_copy(feat_hbm.at[pl.ds(rbase, CHUNK)],
                         feat_bufs.at[b], gsems[b])
        pltpu.async_copy(idx_hbm.at[pl.ds(rbase, CHUNK)],
                         idx_bufs.at[b], gsems[b])

    def wait_gather(b, k):
        rbase = pl.multiple_of((mybase + lo + k) * CHUNK, 8)
        pltpu.make_async_copy(feat_hbm.at[pl.ds(rbase, CHUNK)],
                              feat_bufs.at[b], gsems[b]).wait()
        pltpu.make_async_copy(idx_hbm.at[pl.ds(rbase, CHUNK)],
                              idx_bufs.at[b], gsems[b]).wait()

    for b in range(NBUF):
        @pl.when(b < qs)
        def _prime():
            issue_gather(b, b)

    def superstep(i, carry):
        for b in range(NBUF):
            k = i * NBUF + b

            @pl.when(k < qs)
            def _do():
                wait_gather(b, k)
                for j in range(128 // 16):
                    v = idx_bufs[b, pl.ds(j * 16, 16)] - rowbase
                    oob = (v < 0) | (v >= SH)
                    idx_bufs[b, pl.ds(j * 16, 16)] = jnp.where(oob, SH, v)
                pltpu.sync_copy(feat_bufs.at[b], acc.at[idx_bufs.at[b]],
                                add=True)

                @pl.when(k + NBUF < qs)
                def _prefetch():
                    issue_gather(b, k + NBUF)
        return carry

    lax.fori_loop(0, (q + NBUF - 1) // NBUF, superstep, 0)
    plsc.subcore_barrier()

    obase = pl.multiple_of(c * SH + off, 8)
    pltpu.sync_copy(acc.at[pl.ds(off, 312)], out_hbm.at[pl.ds(obase, 312)])

    @pl.when((s % 8 == 7) & (s < NS - 1))
    def _dump_tail():
        pltpu.sync_copy(acc.at[pl.ds(pl.multiple_of(off + 312, 8), 8)],
                        out_hbm.at[pl.ds(pl.multiple_of(c * SH + off + 312, 8), 8)])


@jax.jit
def kernel(features, structural_indices):
    return _seg_sum_sc(features, structural_indices)
